# baseline (speedup 1.0000x reference)
import functools

import numpy as np
import jax
import jax.numpy as jnp
from jax import lax
from jax.experimental import pallas as pl
from jax.experimental.pallas import tpu as pltpu
from jax.experimental.pallas import tpu_sc as plsc

N = 10000
E = 320000
F = 128
H = 32
C = 64
PROP_RANGE = 3
ALPHA = 1.0
BETA = 0.5

NC = 2
NS = 16
NW = NC * NS

CHUNK = 104
CPT = 104
E_PAD = NW * CPT * CHUNK
NPAD = 10112
ZROWS = NPAD // NS


NBUF = 4
GROUPS = CPT // NBUF


def _sc_scatter_body(p_hbm, rows_hbm, cols_hbm, zeros_hbm, out_hbm,
                     ridx, cidx, gb, acc, ptab, gsem, ssem):
    c = lax.axis_index("c")
    s = lax.axis_index("s")
    w = c * NS + s

    with jax.named_scope("sc_init"):
        pltpu.sync_copy(zeros_hbm.at[pl.ds(s * ZROWS, ZROWS)],
                        acc.at[pl.ds(s * ZROWS, ZROWS)])
        pltpu.sync_copy(p_hbm.at[pl.ds(s * ZROWS, ZROWS)],
                        ptab.at[pl.ds(s * ZROWS, ZROWS)])

        base = w * CPT
        pltpu.sync_copy(rows_hbm.at[pl.ds(base, CPT)], ridx)
        pltpu.sync_copy(cols_hbm.at[pl.ds(base, CPT)], cidx)
        plsc.subcore_barrier()

    with jax.named_scope("sc_edges"):
        for b in range(NBUF):
            pltpu.async_copy(ptab.at[cidx.at[b]], gb.at[b], gsem.at[b])

        def group(i, carry):
            j0 = i * NBUF
            for b in range(NBUF):
                pltpu.make_async_copy(
                    ptab.at[cidx.at[j0 + b]], gb.at[b], gsem.at[b]).wait()
                pltpu.async_copy(gb.at[b], acc.at[ridx.at[j0 + b]],
                                 ssem.at[b], add=True)
            for b in range(NBUF):
                pltpu.make_async_copy(
                    gb.at[b], acc.at[ridx.at[j0 + b]], ssem.at[b]).wait()
                pltpu.async_copy(ptab.at[cidx.at[j0 + NBUF + b]], gb.at[b],
                                 gsem.at[b])
            return carry

        lax.fori_loop(0, GROUPS - 1, group, 0)

        j0 = CPT - NBUF
        for b in range(NBUF):
            pltpu.make_async_copy(
                ptab.at[cidx.at[j0 + b]], gb.at[b], gsem.at[b]).wait()
            pltpu.async_copy(gb.at[b], acc.at[ridx.at[j0 + b]],
                             ssem.at[b], add=True)
        for b in range(NBUF):
            pltpu.make_async_copy(
                gb.at[b], acc.at[ridx.at[j0 + b]], ssem.at[b]).wait()

        plsc.subcore_barrier()

    with jax.named_scope("sc_out"):
        pltpu.sync_copy(acc.at[pl.ds(s * ZROWS, ZROWS)],
                        out_hbm.at[c, pl.ds(s * ZROWS, ZROWS)])


_sc_scatter = pl.kernel(
    _sc_scatter_body,
    out_type=jax.ShapeDtypeStruct((NC, NPAD, C), jnp.float32),
    mesh=plsc.VectorSubcoreMesh(core_axis_name="c", subcore_axis_name="s",
                                num_cores=NC, num_subcores=NS),
    scratch_types=[
        pltpu.VMEM((CPT, CHUNK), jnp.int32),
        pltpu.VMEM((CPT, CHUNK), jnp.int32),
        pltpu.VMEM((NBUF, CHUNK, C), jnp.float32),
        pltpu.VMEM_SHARED((NPAD, C), jnp.float32),
        pltpu.VMEM_SHARED((NPAD, C), jnp.float32),
        pltpu.SemaphoreType.DMA((NBUF,)),
        pltpu.SemaphoreType.DMA((NBUF,)),
    ],
    compiler_params=pltpu.CompilerParams(use_tc_tiling_on_sc=False),
)


ROWS_BLK = 2000


def _mlp_body(x_ref, w1_ref, b1_ref, w2_ref, b2_ref, out_ref):
    h = jnp.dot(x_ref[...], w1_ref[...], preferred_element_type=jnp.float32)
    h = jnp.maximum(h + b1_ref[...], 0.0)
    lg = jnp.dot(h, w2_ref[...], preferred_element_type=jnp.float32)
    lg = lg + b2_ref[...]
    e = jnp.exp(lg - jnp.max(lg, axis=-1, keepdims=True))
    out_ref[...] = e / jnp.sum(e, axis=-1, keepdims=True)


def _mlp(x, W1, b1, W2, b2):
    return pl.pallas_call(
        _mlp_body,
        grid=(N // ROWS_BLK,),
        in_specs=[
            pl.BlockSpec((ROWS_BLK, F), lambda i: (i, 0)),
            pl.BlockSpec((F, H), lambda i: (0, 0)),
            pl.BlockSpec((1, H), lambda i: (0, 0)),
            pl.BlockSpec((H, C), lambda i: (0, 0)),
            pl.BlockSpec((1, C), lambda i: (0, 0)),
        ],
        out_specs=pl.BlockSpec((ROWS_BLK, C), lambda i: (i, 0)),
        out_shape=jax.ShapeDtypeStruct((NPAD, C), jnp.float32),
    )(x, W1, b1.reshape(1, H), W2, b2.reshape(1, C))


def _hop_body(parts_ref, p_ref, y_ref, pnew_ref, ynew_ref):
    t = parts_ref[0] + parts_ref[1] + p_ref[...]
    t = jax.nn.sigmoid(ALPHA * t + BETA)
    pnew_ref[...] = t
    e = jnp.exp(t - jnp.max(t, axis=-1, keepdims=True))
    ynew_ref[...] = y_ref[...] + e / jnp.sum(e, axis=-1, keepdims=True)


def _hop(parts, p, y):
    return pl.pallas_call(
        _hop_body,
        grid=(N // ROWS_BLK,),
        in_specs=[
            pl.BlockSpec((NC, ROWS_BLK, C), lambda i: (0, i, 0)),
            pl.BlockSpec((ROWS_BLK, C), lambda i: (i, 0)),
            pl.BlockSpec((ROWS_BLK, C), lambda i: (i, 0)),
        ],
        out_specs=[
            pl.BlockSpec((ROWS_BLK, C), lambda i: (i, 0)),
            pl.BlockSpec((ROWS_BLK, C), lambda i: (i, 0)),
        ],
        out_shape=[
            jax.ShapeDtypeStruct((NPAD, C), jnp.float32),
            jax.ShapeDtypeStruct((N, C), jnp.float32),
        ],
    )(parts, p, y)


def kernel(x, edge_index, W1, b1, W2, b2):
    rows = edge_index[0]
    cols = edge_index[1]
    pad = E_PAD - E
    trash = jnp.asarray(N + (np.arange(pad) % (NPAD - N)).astype(np.int32))
    rows_p = jnp.concatenate([rows, trash])
    cols_p = jnp.concatenate([cols, jnp.zeros((pad,), jnp.int32)])
    rows2d = rows_p.reshape(NW * CPT, CHUNK)
    cols2d = cols_p.reshape(NW * CPT, CHUNK)
    zeros_pad = jnp.zeros((NPAD, C), jnp.float32)

    p = _mlp(x, W1, b1, W2, b2)
    y = jnp.zeros((N, C), jnp.float32)
    for _ in range(PROP_RANGE):
        parts = _sc_scatter(p, rows2d, cols2d, zeros_pad)
        p, y = _hop(parts, p, y)
    return y


# speedup vs baseline: 1.1728x; 1.1728x over previous
import functools

import numpy as np
import jax
import jax.numpy as jnp
from jax import lax
from jax.experimental import pallas as pl
from jax.experimental.pallas import tpu as pltpu
from jax.experimental.pallas import tpu_sc as plsc

N = 10000
E = 320000
F = 128
H = 32
C = 64
PROP_RANGE = 3
ALPHA = 1.0
BETA = 0.5

NC = 2
NS = 16
NW = NC * NS

CHUNK = 128
CPT = 80
E_PAD = NW * CPT * CHUNK
NPAD = 10112
ZROWS = NPAD // NS


NBUF = 2
GROUPS = CPT // NBUF


def _sc_scatter_body(p_hbm, rows_hbm, cols_hbm, zeros_hbm, out_hbm,
                     ridx, cidx, gb, acc, ptab, gsem, ssem):
    c = lax.axis_index("c")
    s = lax.axis_index("s")
    w = c * NS + s

    with jax.named_scope("sc_init"):
        pltpu.sync_copy(zeros_hbm.at[pl.ds(s * ZROWS, ZROWS)],
                        acc.at[pl.ds(s * ZROWS, ZROWS)])
        pltpu.sync_copy(p_hbm.at[pl.ds(s * ZROWS, ZROWS)],
                        ptab.at[pl.ds(s * ZROWS, ZROWS)])

        base = w * CPT
        pltpu.sync_copy(rows_hbm.at[pl.ds(base, CPT)], ridx)
        pltpu.sync_copy(cols_hbm.at[pl.ds(base, CPT)], cidx)
        plsc.subcore_barrier()

    with jax.named_scope("sc_edges"):
        for b in range(NBUF):
            pltpu.async_copy(ptab.at[cidx.at[b]], gb.at[b], gsem.at[b])

        def group(i, carry):
            j0 = i * NBUF
            for b in range(NBUF):
                pltpu.make_async_copy(
                    ptab.at[cidx.at[j0 + b]], gb.at[b], gsem.at[b]).wait()
                pltpu.async_copy(gb.at[b], acc.at[ridx.at[j0 + b]],
                                 ssem.at[b], add=True)
            for b in range(NBUF):
                pltpu.make_async_copy(
                    gb.at[b], acc.at[ridx.at[j0 + b]], ssem.at[b]).wait()
                pltpu.async_copy(ptab.at[cidx.at[j0 + NBUF + b]], gb.at[b],
                                 gsem.at[b])
            return carry

        lax.fori_loop(0, GROUPS - 1, group, 0)

        j0 = CPT - NBUF
        for b in range(NBUF):
            pltpu.make_async_copy(
                ptab.at[cidx.at[j0 + b]], gb.at[b], gsem.at[b]).wait()
            pltpu.async_copy(gb.at[b], acc.at[ridx.at[j0 + b]],
                             ssem.at[b], add=True)
        for b in range(NBUF):
            pltpu.make_async_copy(
                gb.at[b], acc.at[ridx.at[j0 + b]], ssem.at[b]).wait()

        plsc.subcore_barrier()

    with jax.named_scope("sc_out"):
        pltpu.sync_copy(acc.at[pl.ds(s * ZROWS, ZROWS)],
                        out_hbm.at[c, pl.ds(s * ZROWS, ZROWS)])


_sc_scatter = pl.kernel(
    _sc_scatter_body,
    out_type=jax.ShapeDtypeStruct((NC, NPAD, C), jnp.float32),
    mesh=plsc.VectorSubcoreMesh(core_axis_name="c", subcore_axis_name="s",
                                num_cores=NC, num_subcores=NS),
    scratch_types=[
        pltpu.VMEM((CPT, CHUNK), jnp.int32),
        pltpu.VMEM((CPT, CHUNK), jnp.int32),
        pltpu.VMEM((NBUF, CHUNK, C), jnp.float32),
        pltpu.VMEM_SHARED((NPAD, C), jnp.float32),
        pltpu.VMEM_SHARED((NPAD, C), jnp.float32),
        pltpu.SemaphoreType.DMA((NBUF,)),
        pltpu.SemaphoreType.DMA((NBUF,)),
    ],
    compiler_params=pltpu.CompilerParams(use_tc_tiling_on_sc=False),
)


ROWS_BLK = 2000


def _mlp_body(x_ref, w1_ref, b1_ref, w2_ref, b2_ref, out_ref):
    h = jnp.dot(x_ref[...], w1_ref[...], preferred_element_type=jnp.float32)
    h = jnp.maximum(h + b1_ref[...], 0.0)
    lg = jnp.dot(h, w2_ref[...], preferred_element_type=jnp.float32)
    lg = lg + b2_ref[...]
    e = jnp.exp(lg - jnp.max(lg, axis=-1, keepdims=True))
    out_ref[...] = e / jnp.sum(e, axis=-1, keepdims=True)


def _mlp(x, W1, b1, W2, b2):
    return pl.pallas_call(
        _mlp_body,
        grid=(N // ROWS_BLK,),
        in_specs=[
            pl.BlockSpec((ROWS_BLK, F), lambda i: (i, 0)),
            pl.BlockSpec((F, H), lambda i: (0, 0)),
            pl.BlockSpec((1, H), lambda i: (0, 0)),
            pl.BlockSpec((H, C), lambda i: (0, 0)),
            pl.BlockSpec((1, C), lambda i: (0, 0)),
        ],
        out_specs=pl.BlockSpec((ROWS_BLK, C), lambda i: (i, 0)),
        out_shape=jax.ShapeDtypeStruct((NPAD, C), jnp.float32),
    )(x, W1, b1.reshape(1, H), W2, b2.reshape(1, C))


def _hop_body(parts_ref, p_ref, y_ref, pnew_ref, ynew_ref):
    t = parts_ref[0] + parts_ref[1] + p_ref[...]
    t = jax.nn.sigmoid(ALPHA * t + BETA)
    pnew_ref[...] = t
    e = jnp.exp(t - jnp.max(t, axis=-1, keepdims=True))
    ynew_ref[...] = y_ref[...] + e / jnp.sum(e, axis=-1, keepdims=True)


def _hop(parts, p, y):
    return pl.pallas_call(
        _hop_body,
        grid=(N // ROWS_BLK,),
        in_specs=[
            pl.BlockSpec((NC, ROWS_BLK, C), lambda i: (0, i, 0)),
            pl.BlockSpec((ROWS_BLK, C), lambda i: (i, 0)),
            pl.BlockSpec((ROWS_BLK, C), lambda i: (i, 0)),
        ],
        out_specs=[
            pl.BlockSpec((ROWS_BLK, C), lambda i: (i, 0)),
            pl.BlockSpec((ROWS_BLK, C), lambda i: (i, 0)),
        ],
        out_shape=[
            jax.ShapeDtypeStruct((NPAD, C), jnp.float32),
            jax.ShapeDtypeStruct((N, C), jnp.float32),
        ],
    )(parts, p, y)


def kernel(x, edge_index, W1, b1, W2, b2):
    rows = edge_index[0]
    cols = edge_index[1]
    pad = E_PAD - E
    trash = jnp.asarray(N + (np.arange(pad) % (NPAD - N)).astype(np.int32))
    rows_p = jnp.concatenate([rows, trash])
    cols_p = jnp.concatenate([cols, jnp.zeros((pad,), jnp.int32)])
    rows2d = rows_p.reshape(NW * CPT, CHUNK)
    cols2d = cols_p.reshape(NW * CPT, CHUNK)
    zeros_pad = jnp.zeros((NPAD, C), jnp.float32)

    p = _mlp(x, W1, b1, W2, b2)
    y = jnp.zeros((N, C), jnp.float32)
    for _ in range(PROP_RANGE):
        parts = _sc_scatter(p, rows2d, cols2d, zeros_pad)
        p, y = _hop(parts, p, y)
    return y


# speedup vs baseline: 1.6967x; 1.4468x over previous
import functools

import numpy as np
import jax
import jax.numpy as jnp
from jax import lax
from jax.experimental import pallas as pl
from jax.experimental.pallas import tpu as pltpu
from jax.experimental.pallas import tpu_sc as plsc

N = 10000
E = 320000
F = 128
H = 32
C = 64
PROP_RANGE = 3
ALPHA = 1.0
BETA = 0.5

NC = 2
NS = 16
NW = NC * NS

CHUNK = 128
CPT = 80
E_PAD = NW * CPT * CHUNK
NPAD = 10112
ZROWS = NPAD // NS


NBUF = 2
GROUPS = CPT // NBUF


def _sc_scatter_body(p_hbm, rows_hbm, cols_hbm, zeros_hbm, out_hbm,
                     ridx, cidx, gb, acc, ptab, gsem, ssem):
    c = lax.axis_index("c")
    s = lax.axis_index("s")
    w = c * NS + s

    with jax.named_scope("sc_init"):
        pltpu.sync_copy(zeros_hbm.at[pl.ds(s * ZROWS, ZROWS)],
                        acc.at[pl.ds(s * ZROWS, ZROWS)])
        pltpu.sync_copy(p_hbm.at[pl.ds(s * ZROWS, ZROWS)],
                        ptab.at[pl.ds(s * ZROWS, ZROWS)])

        base = w * CPT
        pltpu.sync_copy(rows_hbm.at[pl.ds(base, CPT)], ridx)
        pltpu.sync_copy(cols_hbm.at[pl.ds(base, CPT)], cidx)
        plsc.subcore_barrier()

    with jax.named_scope("sc_edges"):
        pltpu.async_copy(ptab.at[cidx.at[0]], gb.at[0], gsem.at[0])
        pltpu.async_copy(ptab.at[cidx.at[1]], gb.at[1], gsem.at[1])

        def pair(i, carry):
            j = 2 * i
            pltpu.make_async_copy(
                ptab.at[cidx.at[j]], gb.at[0], gsem.at[0]).wait()
            pltpu.sync_copy(gb.at[0], acc.at[ridx.at[j]], add=True)
            pltpu.async_copy(ptab.at[cidx.at[j + 2]], gb.at[0], gsem.at[0])
            pltpu.make_async_copy(
                ptab.at[cidx.at[j + 1]], gb.at[1], gsem.at[1]).wait()
            pltpu.sync_copy(gb.at[1], acc.at[ridx.at[j + 1]], add=True)
            pltpu.async_copy(ptab.at[cidx.at[j + 3]], gb.at[1], gsem.at[1])
            return carry

        lax.fori_loop(0, CPT // 2 - 1, pair, 0)

        j = CPT - 2
        pltpu.make_async_copy(ptab.at[cidx.at[j]], gb.at[0], gsem.at[0]).wait()
        pltpu.sync_copy(gb.at[0], acc.at[ridx.at[j]], add=True)
        pltpu.make_async_copy(
            ptab.at[cidx.at[j + 1]], gb.at[1], gsem.at[1]).wait()
        pltpu.sync_copy(gb.at[1], acc.at[ridx.at[j + 1]], add=True)

        plsc.subcore_barrier()

    with jax.named_scope("sc_out"):
        pltpu.sync_copy(acc.at[pl.ds(s * ZROWS, ZROWS)],
                        out_hbm.at[c, pl.ds(s * ZROWS, ZROWS)])


_sc_scatter = pl.kernel(
    _sc_scatter_body,
    out_type=jax.ShapeDtypeStruct((NC, NPAD, C), jnp.bfloat16),
    mesh=plsc.VectorSubcoreMesh(core_axis_name="c", subcore_axis_name="s",
                                num_cores=NC, num_subcores=NS),
    scratch_types=[
        pltpu.VMEM((CPT, CHUNK), jnp.int32),
        pltpu.VMEM((CPT, CHUNK), jnp.int32),
        pltpu.VMEM((NBUF, CHUNK, C), jnp.bfloat16),
        pltpu.VMEM_SHARED((NPAD, C), jnp.bfloat16),
        pltpu.VMEM_SHARED((NPAD, C), jnp.bfloat16),
        pltpu.SemaphoreType.DMA((NBUF,)),
        pltpu.SemaphoreType.DMA((NBUF,)),
    ],
    compiler_params=pltpu.CompilerParams(use_tc_tiling_on_sc=False),
)


ROWS_BLK = 2000


def _mlp_body(x_ref, w1_ref, b1_ref, w2_ref, b2_ref, out_ref):
    h = jnp.dot(x_ref[...], w1_ref[...], preferred_element_type=jnp.float32)
    h = jnp.maximum(h + b1_ref[...], 0.0)
    lg = jnp.dot(h, w2_ref[...], preferred_element_type=jnp.float32)
    lg = lg + b2_ref[...]
    e = jnp.exp(lg - jnp.max(lg, axis=-1, keepdims=True))
    out_ref[...] = (e / jnp.sum(e, axis=-1, keepdims=True)).astype(jnp.bfloat16)


def _mlp(x, W1, b1, W2, b2):
    return pl.pallas_call(
        _mlp_body,
        grid=(N // ROWS_BLK,),
        in_specs=[
            pl.BlockSpec((ROWS_BLK, F), lambda i: (i, 0)),
            pl.BlockSpec((F, H), lambda i: (0, 0)),
            pl.BlockSpec((1, H), lambda i: (0, 0)),
            pl.BlockSpec((H, C), lambda i: (0, 0)),
            pl.BlockSpec((1, C), lambda i: (0, 0)),
        ],
        out_specs=pl.BlockSpec((ROWS_BLK, C), lambda i: (i, 0)),
        out_shape=jax.ShapeDtypeStruct((NPAD, C), jnp.bfloat16),
    )(x, W1, b1.reshape(1, H), W2, b2.reshape(1, C))


def _hop_body(parts_ref, p_ref, y_ref, pnew_ref, ynew_ref):
    t = (parts_ref[0].astype(jnp.float32) + parts_ref[1].astype(jnp.float32)
         + p_ref[...].astype(jnp.float32))
    t = jax.nn.sigmoid(ALPHA * t + BETA)
    pnew_ref[...] = t.astype(jnp.bfloat16)
    e = jnp.exp(t - jnp.max(t, axis=-1, keepdims=True))
    ynew_ref[...] = y_ref[...] + e / jnp.sum(e, axis=-1, keepdims=True)


def _hop(parts, p, y):
    return pl.pallas_call(
        _hop_body,
        grid=(N // ROWS_BLK,),
        in_specs=[
            pl.BlockSpec((NC, ROWS_BLK, C), lambda i: (0, i, 0)),
            pl.BlockSpec((ROWS_BLK, C), lambda i: (i, 0)),
            pl.BlockSpec((ROWS_BLK, C), lambda i: (i, 0)),
        ],
        out_specs=[
            pl.BlockSpec((ROWS_BLK, C), lambda i: (i, 0)),
            pl.BlockSpec((ROWS_BLK, C), lambda i: (i, 0)),
        ],
        out_shape=[
            jax.ShapeDtypeStruct((NPAD, C), jnp.bfloat16),
            jax.ShapeDtypeStruct((N, C), jnp.float32),
        ],
    )(parts, p, y)


def kernel(x, edge_index, W1, b1, W2, b2):
    rows = edge_index[0]
    cols = edge_index[1]
    pad = E_PAD - E
    trash = jnp.asarray(N + (np.arange(pad) % (NPAD - N)).astype(np.int32))
    rows_p = jnp.concatenate([rows, trash])
    cols_p = jnp.concatenate([cols, jnp.zeros((pad,), jnp.int32)])
    rows2d = rows_p.reshape(NW * CPT, CHUNK)
    cols2d = cols_p.reshape(NW * CPT, CHUNK)
    zeros_pad = jnp.zeros((NPAD, C), jnp.bfloat16)

    p = _mlp(x, W1, b1, W2, b2)
    y = jnp.zeros((N, C), jnp.float32)
    for _ in range(PROP_RANGE):
        parts = _sc_scatter(p, rows2d, cols2d, zeros_pad)
        p, y = _hop(parts, p, y)
    return y


# speedup vs baseline: 1.7770x; 1.0473x over previous
import functools

import numpy as np
import jax
import jax.numpy as jnp
from jax import lax
from jax.experimental import pallas as pl
from jax.experimental.pallas import tpu as pltpu
from jax.experimental.pallas import tpu_sc as plsc

N = 10000
E = 320000
F = 128
H = 32
C = 64
PROP_RANGE = 3
ALPHA = 1.0
BETA = 0.5

NC = 2
NS = 16
NW = NC * NS

CHUNK = 128
CPT = 80
E_PAD = NW * CPT * CHUNK
NPAD = 10112
ZROWS = NPAD // NS
NPAD2 = NPAD // 2


NBUF = 2
GROUPS = CPT // NBUF


def _sc_scatter_body(p_hbm, rows_hbm, cols_hbm, zeros_hbm, out_hbm,
                     ridx, cidx, gb, acc, ptab, gsem, ssem):
    c = lax.axis_index("c")
    s = lax.axis_index("s")
    w = c * NS + s

    with jax.named_scope("sc_init"):
        pltpu.sync_copy(zeros_hbm.at[pl.ds(s * ZROWS, ZROWS)],
                        acc.at[pl.ds(s * ZROWS, ZROWS)])
        pltpu.sync_copy(p_hbm.at[pl.ds(s * ZROWS, ZROWS)],
                        ptab.at[pl.ds(s * ZROWS, ZROWS)])

        base = w * CPT
        pltpu.sync_copy(rows_hbm.at[pl.ds(base, CPT)], ridx)
        pltpu.sync_copy(cols_hbm.at[pl.ds(base, CPT)], cidx)
        plsc.subcore_barrier()

    with jax.named_scope("sc_edges"):
        pltpu.async_copy(ptab.at[cidx.at[0]], gb.at[0], gsem.at[0])
        pltpu.async_copy(ptab.at[cidx.at[1]], gb.at[1], gsem.at[1])

        def pair(i, carry):
            j = 2 * i
            pltpu.make_async_copy(
                ptab.at[cidx.at[j]], gb.at[0], gsem.at[0]).wait()
            pltpu.sync_copy(gb.at[0], acc.at[ridx.at[j]], add=True)
            pltpu.async_copy(ptab.at[cidx.at[j + 2]], gb.at[0], gsem.at[0])
            pltpu.make_async_copy(
                ptab.at[cidx.at[j + 1]], gb.at[1], gsem.at[1]).wait()
            pltpu.sync_copy(gb.at[1], acc.at[ridx.at[j + 1]], add=True)
            pltpu.async_copy(ptab.at[cidx.at[j + 3]], gb.at[1], gsem.at[1])
            return carry

        lax.fori_loop(0, CPT // 2 - 1, pair, 0)

        j = CPT - 2
        pltpu.make_async_copy(ptab.at[cidx.at[j]], gb.at[0], gsem.at[0]).wait()
        pltpu.sync_copy(gb.at[0], acc.at[ridx.at[j]], add=True)
        pltpu.make_async_copy(
            ptab.at[cidx.at[j + 1]], gb.at[1], gsem.at[1]).wait()
        pltpu.sync_copy(gb.at[1], acc.at[ridx.at[j + 1]], add=True)

        plsc.subcore_barrier()

    with jax.named_scope("sc_out"):
        pltpu.sync_copy(acc.at[pl.ds(s * ZROWS, ZROWS)],
                        out_hbm.at[c, pl.ds(s * ZROWS, ZROWS)])


_sc_scatter = pl.kernel(
    _sc_scatter_body,
    out_type=jax.ShapeDtypeStruct((NC, NPAD, C), jnp.bfloat16),
    mesh=plsc.VectorSubcoreMesh(core_axis_name="c", subcore_axis_name="s",
                                num_cores=NC, num_subcores=NS),
    scratch_types=[
        pltpu.VMEM((CPT, CHUNK), jnp.int32),
        pltpu.VMEM((CPT, CHUNK), jnp.int32),
        pltpu.VMEM((NBUF, CHUNK, C), jnp.bfloat16),
        pltpu.VMEM_SHARED((NPAD, C), jnp.bfloat16),
        pltpu.VMEM_SHARED((NPAD, C), jnp.bfloat16),
        pltpu.SemaphoreType.DMA((NBUF,)),
        pltpu.SemaphoreType.DMA((NBUF,)),
    ],
    compiler_params=pltpu.CompilerParams(use_tc_tiling_on_sc=False),
)


ROWS_BLK = 2000


def _mlp_body(x_ref, w1_ref, b1_ref, w2_ref, b2_ref, out_ref):
    h = jnp.dot(x_ref[...], w1_ref[...], preferred_element_type=jnp.float32)
    h = jnp.maximum(h + b1_ref[...], 0.0)
    lg = jnp.dot(h, w2_ref[...], preferred_element_type=jnp.float32)
    lg = lg + b2_ref[...]
    e = jnp.exp(lg - jnp.max(lg, axis=-1, keepdims=True))
    out_ref[...] = (e / jnp.sum(e, axis=-1, keepdims=True)).astype(jnp.bfloat16)


def _mlp(x, W1, b1, W2, b2):
    return pl.pallas_call(
        _mlp_body,
        grid=(N // ROWS_BLK,),
        in_specs=[
            pl.BlockSpec((ROWS_BLK, F), lambda i: (i, 0)),
            pl.BlockSpec((F, H), lambda i: (0, 0)),
            pl.BlockSpec((1, H), lambda i: (0, 0)),
            pl.BlockSpec((H, C), lambda i: (0, 0)),
            pl.BlockSpec((1, C), lambda i: (0, 0)),
        ],
        out_specs=pl.BlockSpec((ROWS_BLK, C), lambda i: (i, 0)),
        out_shape=jax.ShapeDtypeStruct((NPAD, C), jnp.bfloat16),
    )(x, W1, b1.reshape(1, H), W2, b2.reshape(1, C))


HOP_BLK = NPAD2 // 4


def _softmax64(t):
    e = jnp.exp(t - jnp.max(t, axis=-1, keepdims=True))
    return e / jnp.sum(e, axis=-1, keepdims=True)


def _hop_body(parts_ref, p_ref, y_ref, pnew_ref, ynew_ref):
    t = (parts_ref[0].astype(jnp.float32) + parts_ref[1].astype(jnp.float32)
         + p_ref[...].astype(jnp.float32))
    t = jax.nn.sigmoid(ALPHA * t + BETA)
    pnew_ref[...] = t.astype(jnp.bfloat16)
    sm = jnp.concatenate([_softmax64(t[:, :C]), _softmax64(t[:, C:])], axis=-1)
    ynew_ref[...] = y_ref[...] + sm


def _hop(parts128, p128, y128):
    return pl.pallas_call(
        _hop_body,
        grid=(NPAD2 // HOP_BLK,),
        in_specs=[
            pl.BlockSpec((NC, HOP_BLK, 2 * C), lambda i: (0, i, 0)),
            pl.BlockSpec((HOP_BLK, 2 * C), lambda i: (i, 0)),
            pl.BlockSpec((HOP_BLK, 2 * C), lambda i: (i, 0)),
        ],
        out_specs=[
            pl.BlockSpec((HOP_BLK, 2 * C), lambda i: (i, 0)),
            pl.BlockSpec((HOP_BLK, 2 * C), lambda i: (i, 0)),
        ],
        out_shape=[
            jax.ShapeDtypeStruct((NPAD2, 2 * C), jnp.bfloat16),
            jax.ShapeDtypeStruct((NPAD2, 2 * C), jnp.float32),
        ],
    )(parts128, p128, y128)


def kernel(x, edge_index, W1, b1, W2, b2):
    rows = edge_index[0]
    cols = edge_index[1]
    pad = E_PAD - E
    trash = jnp.asarray(N + (np.arange(pad) % (NPAD - N)).astype(np.int32))
    rows_p = jnp.concatenate([rows, trash])
    cols_p = jnp.concatenate([cols, jnp.zeros((pad,), jnp.int32)])
    rows2d = rows_p.reshape(NW * CPT, CHUNK)
    cols2d = cols_p.reshape(NW * CPT, CHUNK)
    zeros_pad = jnp.zeros((NPAD, C), jnp.bfloat16)

    p = _mlp(x, W1, b1, W2, b2)
    y128 = jnp.zeros((NPAD2, 2 * C), jnp.float32)
    for _ in range(PROP_RANGE):
        parts = _sc_scatter(p, rows2d, cols2d, zeros_pad)
        p128, y128 = _hop(parts.reshape(NC, NPAD2, 2 * C),
                          p.reshape(NPAD2, 2 * C), y128)
        p = p128.reshape(NPAD, C)
    return y128.reshape(NPAD, C)[:N]


# speedup vs baseline: 1.7929x; 1.0090x over previous
import functools

import numpy as np
import jax
import jax.numpy as jnp
from jax import lax
from jax.experimental import pallas as pl
from jax.experimental.pallas import tpu as pltpu
from jax.experimental.pallas import tpu_sc as plsc

N = 10000
E = 320000
F = 128
H = 32
C = 64
PROP_RANGE = 3
ALPHA = 1.0
BETA = 0.5

NC = 2
NS = 16
NW = NC * NS

CHUNK = 128
CPT = 80
E_PAD = NW * CPT * CHUNK
NPAD = 10112
ZROWS = NPAD // NS
NPAD2 = NPAD // 2


NBUF = 2
GROUPS = CPT // NBUF


def _sc_scatter_body(p_hbm, rows_hbm, cols_hbm, zeros_hbm, out_hbm,
                     ridx, cidx, gb, acc, ptab, gsem, ssem):
    c = lax.axis_index("c")
    s = lax.axis_index("s")
    w = c * NS + s

    with jax.named_scope("sc_init"):
        pltpu.sync_copy(p_hbm.at[pl.ds(s * ZROWS, ZROWS)],
                        ptab.at[pl.ds(s * ZROWS, ZROWS)])

        @pl.when(c == 0)
        def _():
            pltpu.sync_copy(p_hbm.at[pl.ds(s * ZROWS, ZROWS)],
                            acc.at[pl.ds(s * ZROWS, ZROWS)])

        @pl.when(c != 0)
        def _():
            pltpu.sync_copy(zeros_hbm.at[pl.ds(s * ZROWS, ZROWS)],
                            acc.at[pl.ds(s * ZROWS, ZROWS)])

        base = w * CPT
        pltpu.sync_copy(rows_hbm.at[pl.ds(base, CPT)], ridx)
        pltpu.sync_copy(cols_hbm.at[pl.ds(base, CPT)], cidx)
        plsc.subcore_barrier()

    with jax.named_scope("sc_edges"):
        pltpu.async_copy(ptab.at[cidx.at[0]], gb.at[0], gsem.at[0])
        pltpu.async_copy(ptab.at[cidx.at[1]], gb.at[1], gsem.at[1])

        def pair(i, carry):
            j = 2 * i
            pltpu.make_async_copy(
                ptab.at[cidx.at[j]], gb.at[0], gsem.at[0]).wait()
            pltpu.sync_copy(gb.at[0], acc.at[ridx.at[j]], add=True)
            pltpu.async_copy(ptab.at[cidx.at[j + 2]], gb.at[0], gsem.at[0])
            pltpu.make_async_copy(
                ptab.at[cidx.at[j + 1]], gb.at[1], gsem.at[1]).wait()
            pltpu.sync_copy(gb.at[1], acc.at[ridx.at[j + 1]], add=True)
            pltpu.async_copy(ptab.at[cidx.at[j + 3]], gb.at[1], gsem.at[1])
            return carry

        lax.fori_loop(0, CPT // 2 - 1, pair, 0)

        j = CPT - 2
        pltpu.make_async_copy(ptab.at[cidx.at[j]], gb.at[0], gsem.at[0]).wait()
        pltpu.sync_copy(gb.at[0], acc.at[ridx.at[j]], add=True)
        pltpu.make_async_copy(
            ptab.at[cidx.at[j + 1]], gb.at[1], gsem.at[1]).wait()
        pltpu.sync_copy(gb.at[1], acc.at[ridx.at[j + 1]], add=True)

        plsc.subcore_barrier()

    with jax.named_scope("sc_out"):
        pltpu.sync_copy(acc.at[pl.ds(s * ZROWS, ZROWS)],
                        out_hbm.at[c, pl.ds(s * ZROWS, ZROWS)])


_sc_scatter = pl.kernel(
    _sc_scatter_body,
    out_type=jax.ShapeDtypeStruct((NC, NPAD, C), jnp.bfloat16),
    mesh=plsc.VectorSubcoreMesh(core_axis_name="c", subcore_axis_name="s",
                                num_cores=NC, num_subcores=NS),
    scratch_types=[
        pltpu.VMEM((CPT, CHUNK), jnp.int32),
        pltpu.VMEM((CPT, CHUNK), jnp.int32),
        pltpu.VMEM((NBUF, CHUNK, C), jnp.bfloat16),
        pltpu.VMEM_SHARED((NPAD, C), jnp.bfloat16),
        pltpu.VMEM_SHARED((NPAD, C), jnp.bfloat16),
        pltpu.SemaphoreType.DMA((NBUF,)),
        pltpu.SemaphoreType.DMA((NBUF,)),
    ],
    compiler_params=pltpu.CompilerParams(use_tc_tiling_on_sc=False),
)


ROWS_BLK = 2000


def _mlp_body(x_ref, w1_ref, b1_ref, w2_ref, b2_ref, out_ref):
    h = jnp.dot(x_ref[...], w1_ref[...], preferred_element_type=jnp.float32)
    h = jnp.maximum(h + b1_ref[...], 0.0)
    lg = jnp.dot(h, w2_ref[...], preferred_element_type=jnp.float32)
    lg = lg + b2_ref[...]
    e = jnp.exp(lg - jnp.max(lg, axis=-1, keepdims=True))
    out_ref[...] = (e / jnp.sum(e, axis=-1, keepdims=True)).astype(jnp.bfloat16)


def _mlp(x, W1, b1, W2, b2):
    return pl.pallas_call(
        _mlp_body,
        grid=(N // ROWS_BLK,),
        in_specs=[
            pl.BlockSpec((ROWS_BLK, F), lambda i: (i, 0)),
            pl.BlockSpec((F, H), lambda i: (0, 0)),
            pl.BlockSpec((1, H), lambda i: (0, 0)),
            pl.BlockSpec((H, C), lambda i: (0, 0)),
            pl.BlockSpec((1, C), lambda i: (0, 0)),
        ],
        out_specs=pl.BlockSpec((ROWS_BLK, C), lambda i: (i, 0)),
        out_shape=jax.ShapeDtypeStruct((NPAD, C), jnp.bfloat16),
    )(x, W1, b1.reshape(1, H), W2, b2.reshape(1, C))


HOP_BLK = NPAD2 // 4


def _softmax64(t):
    e = jnp.exp(t - jnp.max(t, axis=-1, keepdims=True))
    return e / jnp.sum(e, axis=-1, keepdims=True)


def _hop_body(parts_ref, y_ref, pnew_ref, ynew_ref):
    t = (parts_ref[0].astype(jnp.float32) + parts_ref[1].astype(jnp.float32))
    t = jax.nn.sigmoid(ALPHA * t + BETA)
    pnew_ref[...] = t.astype(jnp.bfloat16)
    sm = jnp.concatenate([_softmax64(t[:, :C]), _softmax64(t[:, C:])], axis=-1)
    ynew_ref[...] = y_ref[...] + sm


def _hop(parts128, y128):
    return pl.pallas_call(
        _hop_body,
        grid=(NPAD2 // HOP_BLK,),
        in_specs=[
            pl.BlockSpec((NC, HOP_BLK, 2 * C), lambda i: (0, i, 0)),
            pl.BlockSpec((HOP_BLK, 2 * C), lambda i: (i, 0)),
        ],
        out_specs=[
            pl.BlockSpec((HOP_BLK, 2 * C), lambda i: (i, 0)),
            pl.BlockSpec((HOP_BLK, 2 * C), lambda i: (i, 0)),
        ],
        out_shape=[
            jax.ShapeDtypeStruct((NPAD2, 2 * C), jnp.bfloat16),
            jax.ShapeDtypeStruct((NPAD2, 2 * C), jnp.float32),
        ],
    )(parts128, y128)


def kernel(x, edge_index, W1, b1, W2, b2):
    rows = edge_index[0]
    cols = edge_index[1]
    pad = E_PAD - E
    trash = jnp.asarray(N + (np.arange(pad) % (NPAD - N)).astype(np.int32))
    rows_p = jnp.concatenate([rows, trash])
    cols_p = jnp.concatenate([cols, jnp.zeros((pad,), jnp.int32)])
    rows2d = rows_p.reshape(NW * CPT, CHUNK)
    cols2d = cols_p.reshape(NW * CPT, CHUNK)
    zeros_pad = jnp.zeros((NPAD, C), jnp.bfloat16)

    p = _mlp(x, W1, b1, W2, b2)
    y128 = jnp.zeros((NPAD2, 2 * C), jnp.float32)
    for _ in range(PROP_RANGE):
        parts = _sc_scatter(p, rows2d, cols2d, zeros_pad)
        p128, y128 = _hop(parts.reshape(NC, NPAD2, 2 * C), y128)
        p = p128.reshape(NPAD, C)
    return y128.reshape(NPAD, C)[:N]
